# baseline (speedup 1.0000x reference)
import jax
import jax.numpy as jnp
from jax import lax
from jax.experimental import pallas as pl
from jax.experimental.pallas import tpu as pltpu
from jax.experimental.pallas import tpu_sc as plsc

N = 10000
D = 128
L1 = 64
E = 320000

NC, NS = 2, 16
NW = NC * NS
CHUNK = 128
K1 = 160
K2 = 80
E_PAD = NS * K1 * CHUNK
N_PAD = 10112
RPT = N_PAD // NS
NBUF = 8

_f32 = jnp.float32


def _conv1_body(p_ref, x_ref, wr1_ref, b1_ref, wo1_ref, x1_ref):
    z = (jnp.dot(p_ref[0, :N, :], wr1_ref[:L1, :], preferred_element_type=_f32)
         + jnp.dot(p_ref[1, :N, :], wr1_ref[L1:, :], preferred_element_type=_f32)
         + b1_ref[...]
         + jnp.dot(x_ref[...], wo1_ref[...], preferred_element_type=_f32))
    x1_ref[...] = jnp.where(z >= 0, z, 0.2 * z)


def _final_body(p2_ref, x1_ref, wr2_ref, b2_ref, wo2_ref, x_ref, o_ref):
    aggr2 = p2_ref[0, :N, :] + p2_ref[1, :N, :]
    gate = (jnp.dot(aggr2, wr2_ref[...], preferred_element_type=_f32)
            + b2_ref[0, 0]
            + jnp.dot(x1_ref[...], wo2_ref[...], preferred_element_type=_f32))
    m = jnp.max(gate)
    e = jnp.exp(gate - m)
    w = e / jnp.sum(e)
    o_ref[...] = jnp.sum(w * x_ref[...], axis=0, keepdims=True)


def _make_seg_body(kt, split_cols):
    n_full, rem = divmod(RPT, CHUNK)

    def body(t_h, pk_h, out_h, acc_s, pk, sbuf, dbuf, rows, sems):
        c = lax.axis_index("c")
        s = lax.axis_index("s")
        stage = rows.at[0]

        def _z(i, _):
            stage[i // 4, pl.ds((i % 4) * 16, 16)] = jnp.zeros((16,), _f32)
            return 0
        lax.fori_loop(0, CHUNK * 4, _z, 0)
        for kk in range(n_full):
            pltpu.sync_copy(stage, acc_s.at[pl.ds(s * RPT + kk * CHUNK, CHUNK)])
        if rem:
            pltpu.sync_copy(stage.at[pl.ds(0, rem)],
                            acc_s.at[pl.ds(s * RPT + n_full * CHUNK, rem)])
        plsc.subcore_barrier()

        row_sel = s if split_cols else c * NS + s
        pltpu.sync_copy(pk_h.at[row_sel], pk)
        base = c * N if split_cols else 0

        def _unpack(j, b):
            for q in range(CHUNK // 16):
                v = pk[j, pl.ds(q * 16, 16)]
                sbuf[b, pl.ds(q * 16, 16)] = (v & 0xFFFF) + base
                dbuf[b, pl.ds(q * 16, 16)] = lax.shift_right_logical(v, 16)

        for b in range(NBUF):
            _unpack(b, b)
            pltpu.async_copy(t_h.at[sbuf.at[b]], rows.at[b], sems.at[b])

        def _group(g, _):
            for b in range(NBUF):
                j = g * NBUF + b
                pltpu.make_async_copy(t_h.at[sbuf.at[b]], rows.at[b],
                                      sems.at[b]).wait()
                pltpu.sync_copy(rows.at[b], acc_s.at[dbuf.at[b]], add=True)

                @pl.when(j + NBUF < kt)
                def _prefetch():
                    _unpack(j + NBUF, b)
                    pltpu.async_copy(t_h.at[sbuf.at[b]], rows.at[b], sems.at[b])
            return 0
        lax.fori_loop(0, kt // NBUF, _group, 0)
        plsc.subcore_barrier()

        for kk in range(n_full):
            pltpu.sync_copy(acc_s.at[pl.ds(s * RPT + kk * CHUNK, CHUNK)], stage)
            pltpu.sync_copy(stage, out_h.at[c, pl.ds(s * RPT + kk * CHUNK, CHUNK)])
        if rem:
            pltpu.sync_copy(acc_s.at[pl.ds(s * RPT + n_full * CHUNK, rem)],
                            stage.at[pl.ds(0, rem)])
            pltpu.sync_copy(stage.at[pl.ds(0, rem)],
                            out_h.at[c, pl.ds(s * RPT + n_full * CHUNK, rem)])

    return body


def _make_seg(kt, split_cols, n_pk_rows):
    return pl.kernel(
        _make_seg_body(kt, split_cols),
        out_type=jax.ShapeDtypeStruct((NC, N_PAD, L1), _f32),
        mesh=plsc.VectorSubcoreMesh(core_axis_name="c", subcore_axis_name="s"),
        scratch_types=[
            pltpu.VMEM_SHARED((N_PAD, L1), _f32),
            pltpu.VMEM((kt, CHUNK), jnp.int32),
            pltpu.VMEM((NBUF, CHUNK), jnp.int32),
            pltpu.VMEM((NBUF, CHUNK), jnp.int32),
            pltpu.VMEM((NBUF, CHUNK, L1), _f32),
            pltpu.SemaphoreType.DMA((NBUF,)),
        ],
        compiler_params=pltpu.CompilerParams(use_tc_tiling_on_sc=False),
    )


_seg128 = _make_seg(K1, True, NS)
_seg64 = _make_seg(K2, False, NW)

_conv1 = pl.pallas_call(
    _conv1_body,
    out_shape=jax.ShapeDtypeStruct((N, L1), _f32),
)

_final = pl.pallas_call(
    _final_body,
    out_shape=jax.ShapeDtypeStruct((1, D), _f32),
)


def kernel(x, adj_t, W_rel1, b_rel1, W_root1, W_rel2, b_rel2, W_root2):
    src = adj_t[0].astype(jnp.int32)
    dst = adj_t[1].astype(jnp.int32)
    pad = E_PAD - E
    packed = jnp.concatenate(
        [src | (dst << 16), jnp.full((pad,), N << 16, jnp.int32)])
    pk1 = packed.reshape(NS, K1, CHUNK)
    pk2 = packed.reshape(NW, K2, CHUNK)
    xt = jnp.concatenate([x[:, :L1], x[:, L1:]], axis=0)

    parts = _seg128(xt, pk1)
    x1 = _conv1(parts, x, W_rel1, b_rel1.reshape(1, L1), W_root1)
    parts2 = _seg64(x1, pk2)
    out = _final(parts2, x1, W_rel2, b_rel2.reshape(1, 1), W_root2, x)
    return out
